# baseline (speedup 1.0000x reference)
import functools

import jax
import jax.numpy as jnp
from jax import lax
from jax.experimental import pallas as pl
from jax.experimental.pallas import tpu as pltpu
from jax.experimental.pallas import tpu_sc as plsc

_B, _CIMG, _H, _W = 4, 256, 52, 52
_HW = _H * _W
_CLID = 1024
_HID = 256
_NPTS = 32768
_EPS = 1e-5
_NW = 32
_PW = (_B * _NPTS) // _NW
_GROUPS = _PW // 16
_WPB = _NW // _B


def _sc_mask_body(coords_hbm, out_hbm, pts_v, mask_v):
    wid = lax.axis_index("s") * 2 + lax.axis_index("c")

    def _zero(i, c):
        mask_v[pl.ds(i * 16, 16)] = jnp.zeros((16,), jnp.float32)
        return c

    lax.fori_loop(0, _HW // 16, _zero, 0)

    pltpu.sync_copy(coords_hbm.at[wid], pts_v)

    lane = lax.iota(jnp.int32, 16)
    ones = jnp.ones((16,), jnp.float32)

    def _body(i, c):
        base = i * 32
        ex = lane * 2 + base
        xs = plsc.load_gather(pts_v, [ex])
        ys = plsc.load_gather(pts_v, [ex + 1])
        qx = xs * 52.0 / 416.0
        qy = ys * 52.0 / 416.0
        cx = qx.astype(jnp.int32)
        cy = qy.astype(jnp.int32)
        valid = (xs >= 0.0) & (ys >= 0.0) & (cx < _W) & (cy < _H)
        pix = cy * _W + cx
        plsc.store_scatter(mask_v, [pix], ones, mask=valid)
        return c

    lax.fori_loop(0, _GROUPS, _body, 0)
    pltpu.sync_copy(mask_v, out_hbm.at[wid])


@functools.lru_cache(maxsize=1)
def _sc_mask():
    return functools.partial(
        pl.kernel,
        out_type=jax.ShapeDtypeStruct((_NW, _HW), jnp.float32),
        mesh=plsc.VectorSubcoreMesh(core_axis_name="c", subcore_axis_name="s"),
        compiler_params=pltpu.CompilerParams(needs_layout_passes=False),
        scratch_types=[
            pltpu.VMEM((_PW * 2,), jnp.float32),
            pltpu.VMEM((_HW,), jnp.float32),
        ],
    )(_sc_mask_body)


def _phase1_body(img_ref, lid_ref, m32_ref, wimg_ref, bimg_ref, wlid_ref,
                 blid_ref, w1a_ref, w1b_ref, b1_ref, wd_ref, b2d_ref,
                 wt_ref, bf_ref, g_ref, st_ref):
    b = pl.program_id(0)
    x = img_ref[0]

    img_red = jnp.dot(wimg_ref[...], x,
                      preferred_element_type=jnp.float32) + bimg_ref[...]

    V = lax.dot_general(wlid_ref[...], lid_ref[...],
                        (((1,), (1,)), ((), ())),
                        preferred_element_type=jnp.float32)
    oh = (lax.broadcasted_iota(jnp.int32, (1, _B), 1) == b).astype(jnp.float32)
    v_b = jnp.sum(V * oh, axis=1, keepdims=True)

    occ = (jnp.sum(m32_ref[...], axis=0, keepdims=True) > 0.0
           ).astype(jnp.float32)
    lid_red = v_b * occ + blid_ref[...]

    a1 = jnp.dot(w1a_ref[...], img_red, preferred_element_type=jnp.float32)
    a1 = a1 + jnp.dot(w1b_ref[...], lid_red,
                      preferred_element_type=jnp.float32) + b1_ref[...]
    a1 = jnp.maximum(a1, 0.0)

    d = jnp.dot(wd_ref[...], a1,
                preferred_element_type=jnp.float32) + b2d_ref[...]
    iw = 1.0 / (1.0 + jnp.exp(-d))
    fused = img_red * iw + lid_red * (1.0 - iw)

    zpad = jnp.zeros((_HID, 53), jnp.float32)
    padded = jnp.concatenate([zpad, fused, zpad], axis=1)
    p = lax.broadcasted_iota(jnp.int32, (1, _HW), 1)
    px = p % _W
    py = p // _W
    acc = bf_ref[...] + jnp.zeros((_HID, _HW), jnp.float32)
    k = 0
    for dy in (-1, 0, 1):
        for dx in (-1, 0, 1):
            off = dy * _W + dx
            sl = padded[:, 53 + off:53 + off + _HW]
            t = jnp.dot(wt_ref[k], sl, preferred_element_type=jnp.float32)
            m = ((px + dx >= 0) & (px + dx < _W) &
                 (py + dy >= 0) & (py + dy < _H)).astype(jnp.float32)
            acc = acc + t * m
            k += 1

    g_ref[0] = acc
    s1 = jnp.sum(acc, axis=1, keepdims=True)
    s2 = jnp.sum(acc * acc, axis=1, keepdims=True)
    contrib = jnp.concatenate([s1, s2], axis=1)

    @pl.when(b == 0)
    def _():
        st_ref[...] = contrib

    @pl.when(b != 0)
    def _():
        st_ref[...] += contrib


def _phase2_body(g_ref, st_ref, img_ref, gam_ref, bet_ref, wout_ref, bout_ref,
                 out_ref):
    g = g_ref[0]
    inv_n = 1.0 / float(_B * _HW)
    mean = st_ref[:, 0:1] * inv_n
    var = st_ref[:, 1:2] * inv_n - mean * mean
    scale = gam_ref[...] * lax.rsqrt(var + _EPS)
    y = jnp.maximum((g - mean) * scale + bet_ref[...], 0.0)
    out = jnp.dot(wout_ref[...], y,
                  preferred_element_type=jnp.float32)
    out_ref[0] = out + bout_ref[...] + img_ref[0]


def _full(shape):
    return pl.BlockSpec(shape, lambda b: tuple(0 for _ in shape))


_phase1 = pl.pallas_call(
    _phase1_body,
    grid=(_B,),
    in_specs=[
        pl.BlockSpec((1, _CIMG, _HW), lambda b: (b, 0, 0)),
        _full((_B, _CLID)),
        pl.BlockSpec((_WPB, _HW), lambda b: (b, 0)),
        _full((_HID, _CIMG)),
        _full((_HID, 1)),
        _full((_HID, _CLID)),
        _full((_HID, 1)),
        _full((_HID, _HID)),
        _full((_HID, _HID)),
        _full((_HID, 1)),
        _full((1, _HID)),
        _full((1, 1)),
        _full((9, _HID, _HID)),
        _full((_HID, 1)),
    ],
    out_specs=[
        pl.BlockSpec((1, _HID, _HW), lambda b: (b, 0, 0)),
        pl.BlockSpec((_HID, 2), lambda b: (0, 0)),
    ],
    out_shape=[
        jax.ShapeDtypeStruct((_B, _HID, _HW), jnp.float32),
        jax.ShapeDtypeStruct((_HID, 2), jnp.float32),
    ],
)

_phase2 = pl.pallas_call(
    _phase2_body,
    grid=(_B,),
    in_specs=[
        pl.BlockSpec((1, _HID, _HW), lambda b: (b, 0, 0)),
        _full((_HID, 2)),
        pl.BlockSpec((1, _CIMG, _HW), lambda b: (b, 0, 0)),
        _full((_HID, 1)),
        _full((_HID, 1)),
        _full((_CIMG, _HID)),
        _full((_CIMG, 1)),
    ],
    out_specs=pl.BlockSpec((1, _CIMG, _HW), lambda b: (b, 0, 0)),
    out_shape=jax.ShapeDtypeStruct((_B, _CIMG, _HW), jnp.float32),
)


def kernel(image_features, lidar_features, point_img_coords,
           w_img_red, b_img_red, w_lid_red, b_lid_red,
           w_att1, b_att1, w_att2, b_att2,
           w_fuse, b_fuse, bn_gamma, bn_beta, w_out, b_out):
    coords32 = point_img_coords.reshape(_NW, _PW * 2)
    mask32 = jnp.ones((_NW, _HW), jnp.float32) * coords32[0, 0]

    img_r = image_features.reshape(_B, _CIMG, _HW)
    col = lambda v: v.reshape(-1, 1)
    w1a = w_att1[:, :_HID]
    w1b = w_att1[:, _HID:]
    wd = (w_att2[0] - w_att2[1]).reshape(1, _HID)
    b2d = (b_att2[0] - b_att2[1]).reshape(1, 1)
    w_taps = jnp.zeros((9, _HID, _HID), jnp.float32) + w_fuse[0, 0, 0, 0]

    g, stats = _phase1(img_r, lidar_features, mask32,
                       w_img_red, col(b_img_red), w_lid_red, col(b_lid_red),
                       w1a, w1b, col(b_att1), wd, b2d, w_taps, col(b_fuse))
    out = g + stats[0, 0]
    return out.reshape(_B, _CIMG, _H, _W)


# speedup vs baseline: 2.4463x; 2.4463x over previous
import functools

import jax
import jax.numpy as jnp
from jax import lax
from jax.experimental import pallas as pl
from jax.experimental.pallas import tpu as pltpu
from jax.experimental.pallas import tpu_sc as plsc

_B, _CIMG, _H, _W = 4, 256, 52, 52
_HW = _H * _W
_CLID = 1024
_HID = 256
_NPTS = 32768
_EPS = 1e-5
_NW = 32
_PW = (_B * _NPTS) // _NW
_GROUPS = _PW // 16
_WPB = _NW // _B


def _sc_mask_body(coords_hbm, out_hbm, xs_v, ys_v, mask_v):
    wid = lax.axis_index("s") * 2 + lax.axis_index("c")
    b = wid // _WPB
    base = (wid % _WPB) * _PW

    def _zero(i, c):
        mask_v[pl.ds(i * 16, 16)] = jnp.zeros((16,), jnp.float32)
        return c

    lax.fori_loop(0, _HW // 16, _zero, 0)

    pltpu.sync_copy(coords_hbm.at[2 * b, pl.ds(base, _PW)], xs_v)
    pltpu.sync_copy(coords_hbm.at[2 * b + 1, pl.ds(base, _PW)], ys_v)

    ones = jnp.ones((16,), jnp.float32)

    def _body(i, c):
        xs = xs_v[pl.ds(i * 16, 16)]
        ys = ys_v[pl.ds(i * 16, 16)]
        qx = xs * 52.0 / 416.0
        qy = ys * 52.0 / 416.0
        cx = qx.astype(jnp.int32)
        cy = qy.astype(jnp.int32)
        valid = (xs >= 0.0) & (ys >= 0.0) & (cx < _W) & (cy < _H)
        pix = cy * _W + cx
        plsc.store_scatter(mask_v, [pix], ones, mask=valid)
        return c

    lax.fori_loop(0, _GROUPS, _body, 0)
    pltpu.sync_copy(mask_v, out_hbm.at[wid])


@functools.lru_cache(maxsize=1)
def _sc_mask():
    return functools.partial(
        pl.kernel,
        out_type=jax.ShapeDtypeStruct((_NW, _HW), jnp.float32),
        mesh=plsc.VectorSubcoreMesh(core_axis_name="c", subcore_axis_name="s"),
        compiler_params=pltpu.CompilerParams(needs_layout_passes=False),
        scratch_types=[
            pltpu.VMEM((_PW,), jnp.float32),
            pltpu.VMEM((_PW,), jnp.float32),
            pltpu.VMEM((_HW,), jnp.float32),
        ],
    )(_sc_mask_body)


def _fused_body(img_ref, lid_ref, m32_ref, wimg_ref, wlid_ref,
                w1_ref, w2_ref, b2_ref, bc_ref, wt_ref, wout_ref,
                out_ref, g_s, st_s, img_s):
    i = pl.program_id(0)
    b = i % _B

    @pl.when(i < _B)
    def _phase1():
        _phase1_work(img_ref, lid_ref, m32_ref, wimg_ref, wlid_ref,
                     w1_ref, w2_ref, b2_ref, bc_ref, wt_ref, g_s, st_s,
                     img_s, b)

    @pl.when(i >= _B)
    def _phase2():
        g = g_s[b]
        x_res = img_s[b]
        inv_n = 1.0 / float(_B * _HW)
        mean = st_s[:, 0:1] * inv_n
        var = st_s[:, 1:2] * inv_n - mean * mean
        scale = bc_ref[:, 4:5] * lax.rsqrt(var + _EPS)
        y = jnp.maximum((g - mean) * scale + bc_ref[:, 5:6], 0.0)
        out = jnp.dot(wout_ref[...].astype(jnp.bfloat16), y.astype(jnp.bfloat16),
                      preferred_element_type=jnp.float32) + bc_ref[:, 6:7]
        out_ref[0] = out + x_res


def _phase1_work(img_ref, lid_ref, m32_ref, wimg_ref, wlid_ref,
                 w1_ref, w2_ref, b2_ref, bc_ref, wt_ref, g_ref, st_ref,
                 img_s, b):
    x = img_ref[0]
    img_s[b] = x
    xb = x.astype(jnp.bfloat16)
    bimg = bc_ref[:, 0:1]
    blid = bc_ref[:, 1:2]
    b1 = bc_ref[:, 2:3]
    bfuse = bc_ref[:, 3:4]

    img_red = jnp.dot(wimg_ref[...].astype(jnp.bfloat16), xb,
                      preferred_element_type=jnp.float32) + bimg

    V = lax.dot_general(wlid_ref[...], lid_ref[...],
                        (((1,), (1,)), ((), ())),
                        preferred_element_type=jnp.float32)
    oh = (lax.broadcasted_iota(jnp.int32, (1, _B), 1) == b).astype(jnp.float32)
    v_b = jnp.sum(V * oh, axis=1, keepdims=True)

    occ = (jnp.sum(m32_ref[...], axis=0, keepdims=True) > 0.0
           ).astype(jnp.float32)
    lid_red = v_b * occ + blid

    irb = img_red.astype(jnp.bfloat16)
    lrb = lid_red.astype(jnp.bfloat16)
    w1b16 = w1_ref[...].astype(jnp.bfloat16)
    a1 = jnp.dot(w1b16[:, :_HID], irb, preferred_element_type=jnp.float32)
    a1 = a1 + jnp.dot(w1b16[:, _HID:], lrb,
                      preferred_element_type=jnp.float32) + b1
    a1 = jnp.maximum(a1, 0.0)

    wd = (w2_ref[0:1, :] - w2_ref[1:2, :]).astype(jnp.bfloat16)
    b2d = b2_ref[:, 0:1] - b2_ref[:, 1:2]
    d = jnp.dot(wd, a1.astype(jnp.bfloat16),
                preferred_element_type=jnp.float32) + b2d
    iw = 1.0 / (1.0 + jnp.exp(-d))
    iwb = iw.astype(jnp.bfloat16)
    fb = irb * iwb + lrb * (jnp.bfloat16(1.0) - iwb)

    zpad = jnp.zeros((_HID, 53), jnp.bfloat16)
    padded = jnp.concatenate([zpad, fb, zpad], axis=1)
    p = lax.broadcasted_iota(jnp.int32, (1, _HW), 1)
    px = p % _W
    py = p // _W
    acc = bfuse + jnp.zeros((_HID, _HW), jnp.float32)
    k = 0
    for dy in (-1, 0, 1):
        for dx in (-1, 0, 1):
            off = dy * _W + dx
            sl = padded[:, 53 + off:53 + off + _HW]
            t = jnp.dot(wt_ref[k].astype(jnp.bfloat16), sl,
                        preferred_element_type=jnp.float32)
            m = ((px + dx >= 0) & (px + dx < _W) &
                 (py + dy >= 0) & (py + dy < _H)).astype(jnp.float32)
            acc = acc + t * m
            k += 1

    g_ref[b] = acc
    s1 = jnp.sum(acc, axis=1, keepdims=True)
    s2 = jnp.sum(acc * acc, axis=1, keepdims=True)
    contrib = jnp.concatenate([s1, s2], axis=1)

    @pl.when(b == 0)
    def _():
        st_ref[...] = contrib

    @pl.when(b != 0)
    def _():
        st_ref[...] += contrib


def _full(shape):
    return pl.BlockSpec(shape, lambda b: tuple(0 for _ in shape))


def _imap(i):
    return (jnp.where(i < _B, i, _B - 1), 0, 0)


_fused = pl.pallas_call(
    _fused_body,
    grid=(2 * _B,),
    in_specs=[
        pl.BlockSpec((1, _CIMG, _HW), _imap),
        _full((_B, _CLID)),
        pl.BlockSpec((_WPB, _HW), lambda i: (jnp.where(i < _B, i, _B - 1), 0)),
        _full((_HID, _CIMG)),
        _full((_HID, _CLID)),
        _full((_HID, 2 * _HID)),
        _full((2, _HID)),
        _full((1, 2)),
        _full((_HID, 7)),
        _full((9, _HID, _HID)),
        _full((_CIMG, _HID)),
    ],
    out_specs=pl.BlockSpec((1, _CIMG, _HW),
                           lambda i: (jnp.where(i < _B, 0, i - _B), 0, 0)),
    out_shape=jax.ShapeDtypeStruct((_B, _CIMG, _HW), jnp.float32),
    compiler_params=pltpu.CompilerParams(vmem_limit_bytes=100 * 1024 * 1024),
    scratch_shapes=[
        pltpu.VMEM((_B, _HID, _HW), jnp.float32),
        pltpu.VMEM((_HID, 2), jnp.float32),
        pltpu.VMEM((_B, _CIMG, _HW), jnp.float32),
    ],
)


def kernel(image_features, lidar_features, point_img_coords,
           w_img_red, b_img_red, w_lid_red, b_lid_red,
           w_att1, b_att1, w_att2, b_att2,
           w_fuse, b_fuse, bn_gamma, bn_beta, w_out, b_out):
    coords8 = jnp.transpose(point_img_coords, (0, 2, 1)).reshape(2 * _B, _NPTS)
    mask32 = _sc_mask()(coords8)

    bcols = jnp.stack([b_img_red, b_lid_red, b_att1, b_fuse,
                       bn_gamma, bn_beta, b_out], axis=1)
    w_taps = jnp.transpose(w_fuse, (2, 3, 0, 1)).reshape(9, _HID, _HID)

    img_flat = image_features.reshape(_B, _CIMG, _HW)
    out = _fused(img_flat, lidar_features, mask32,
                 w_img_red, w_lid_red, w_att1, w_att2,
                 b_att2.reshape(1, 2), bcols, w_taps, w_out)
    return out.reshape(_B, _CIMG, _H, _W)
